# baseline (speedup 1.0000x reference)
import functools

import jax
import jax.numpy as jnp
from jax import lax
from jax.experimental import pallas as pl
from jax.experimental.pallas import tpu as pltpu
from jax.experimental.pallas import tpu_sc as plsc

_V = 100000
_E = 64
_O = 128
_B = 16384
_F = 26
_BF = _B * _F

_NHALF = 2
_FH = _BF // _NHALF

_NC = 2
_NS = 16
_NW = _NC * _NS
_FPW = _FH // _NW
_CH = 128
_NCHUNK = _FPW // _CH
_NBUF = 4

_NBB = 2048
_FPH = _F // _NHALF
_NBGRID = _B // _NBB

_sc_mesh = plsc.VectorSubcoreMesh(core_axis_name="c", subcore_axis_name="s")


def _make_sc_gather(half):
    @functools.partial(
        pl.kernel,
        mesh=_sc_mesh,
        out_type=jax.ShapeDtypeStruct((_FH, _O), jnp.float32),
        scratch_types=[
            pltpu.VMEM((_FPW,), jnp.int32),
            pltpu.VMEM((_NBUF, _CH, _O), jnp.float32),
            pltpu.SemaphoreType.DMA,
            pltpu.SemaphoreType.DMA,
        ],
        name=f"sc_gather_h{half}",
    )
    def _sc_gather(tab_hbm, idx_hbm, e2_hbm, idx_v, rows_v, gsem, ssem):
        wid = lax.axis_index("s") * _NC + lax.axis_index("c")
        fbase = wid * _FPW
        pltpu.sync_copy(idx_hbm.at[pl.ds(half * _FH + fbase, _FPW)], idx_v)

        def outer(jo, carry):
            j0 = jo * _NBUF
            gathers = []
            for bi in range(_NBUF):
                ids = idx_v.at[pl.ds((j0 + bi) * _CH, _CH)]
                gathers.append(pltpu.async_copy(tab_hbm.at[ids], rows_v.at[bi], gsem))
            scatters = []
            for bi in range(_NBUF):
                gathers[bi].wait()
                dst = e2_hbm.at[pl.ds(fbase + (j0 + bi) * _CH, _CH)]
                scatters.append(pltpu.async_copy(rows_v.at[bi], dst, ssem))
            for s in scatters:
                s.wait()
            return carry

        lax.fori_loop(0, _NCHUNK // _NBUF, outer, 0)

    return _sc_gather


_sc_gather_halves = [_make_sc_gather(h) for h in range(_NHALF)]


def _mm_body(e2_ref, w_ref, b_ref, out_ref):
    a = lax.dot_general(
        e2_ref[:].astype(jnp.bfloat16), w_ref[:],
        dimension_numbers=(((1,), (0,)), ((), ())),
        preferred_element_type=jnp.float32,
    )
    out_ref[:] = jnp.maximum(a + b_ref[0], 0.0).reshape(1, _NBB, _O)


def _tc_matmul(e2_h, W, b2d, half, prev=None):
    kwargs = {}
    operands = [e2_h, W, b2d]
    in_specs = [
        pl.BlockSpec((_NBB, _O), lambda f, i: (f * _NBGRID + i, 0)),
        pl.BlockSpec((_O, _O), lambda f, i: (0, 0)),
        pl.BlockSpec((1, _O), lambda f, i: (0, 0)),
    ]
    if prev is not None:
        operands.append(prev)
        kwargs["input_output_aliases"] = {3: 0}
        in_specs.append(pl.BlockSpec((1, 8, _O), lambda f, i: (0, 0, 0)))

    def body(*refs):
        _mm_body(refs[0], refs[1], refs[2], refs[-1])

    return pl.pallas_call(
        body,
        grid=(_FPH, _NBGRID),
        in_specs=in_specs,
        out_specs=pl.BlockSpec(
            (1, _NBB, _O), lambda f, i, h=half: (f + h * _FPH, i, 0)
        ),
        out_shape=jax.ShapeDtypeStruct((_F, _B, _O), jnp.float32),
        **kwargs,
    )(*operands)


def kernel(x, emb_table, W, b):
    idx = x.astype(jnp.int32).T.reshape(_BF)
    tab128 = jnp.pad(emb_table, ((0, 0), (0, _O - _E)))
    wbf = jnp.concatenate([W.T, jnp.zeros((_O - _E, _O), W.dtype)], axis=0)
    wbf = wbf.astype(jnp.bfloat16)
    b2d = b.reshape(1, _O)
    e2 = [_sc_gather_halves[h](tab128, idx) for h in range(_NHALF)]
    out = _tc_matmul(e2[0], wbf, b2d, 0)
    for h in range(1, _NHALF):
        out = _tc_matmul(e2[h], wbf, b2d, h, prev=out)
    return jnp.transpose(out, (1, 0, 2))


# speedup vs baseline: 1.1489x; 1.1489x over previous
import functools

import jax
import jax.numpy as jnp
from jax import lax
from jax.experimental import pallas as pl
from jax.experimental.pallas import tpu as pltpu
from jax.experimental.pallas import tpu_sc as plsc

_V = 100000
_E = 64
_O = 128
_B = 16384
_F = 26
_BF = _B * _F

_NHALF = 2
_FH = _BF // _NHALF

_NC = 2
_NS = 16
_NW = _NC * _NS
_FPW = _FH // _NW
_CH = 128
_NCHUNK = _FPW // _CH
_NBUF = 4

_NBB = 4096
_FPH = _F // _NHALF
_NBGRID = _B // _NBB

_sc_mesh = plsc.VectorSubcoreMesh(core_axis_name="c", subcore_axis_name="s")


def _make_sc_gather(half):
    @functools.partial(
        pl.kernel,
        mesh=_sc_mesh,
        out_type=jax.ShapeDtypeStruct((_FH, _O), jnp.float32),
        scratch_types=[
            pltpu.VMEM((_FPW,), jnp.int32),
            pltpu.VMEM((_NBUF, _CH, _O), jnp.float32),
            pltpu.SemaphoreType.DMA,
            pltpu.SemaphoreType.DMA,
        ],
        name=f"sc_gather_h{half}",
    )
    def _sc_gather(tab_hbm, idx_hbm, e2_hbm, idx_v, rows_v, gsem, ssem):
        wid = lax.axis_index("s") * _NC + lax.axis_index("c")
        fbase = wid * _FPW
        pltpu.sync_copy(idx_hbm.at[pl.ds(half * _FH + fbase, _FPW)], idx_v)

        def outer(jo, carry):
            j0 = jo * _NBUF
            gathers = []
            for bi in range(_NBUF):
                ids = idx_v.at[pl.ds((j0 + bi) * _CH, _CH)]
                gathers.append(pltpu.async_copy(tab_hbm.at[ids], rows_v.at[bi], gsem))
            scatters = []
            for bi in range(_NBUF):
                gathers[bi].wait()
                dst = e2_hbm.at[pl.ds(fbase + (j0 + bi) * _CH, _CH)]
                scatters.append(pltpu.async_copy(rows_v.at[bi], dst, ssem))
            for s in scatters:
                s.wait()
            return carry

        lax.fori_loop(0, _NCHUNK // _NBUF, outer, 0)

    return _sc_gather


_sc_gather_halves = [_make_sc_gather(h) for h in range(_NHALF)]


def _mm_body(e2_ref, w_ref, b_ref, out_ref):
    a = lax.dot_general(
        e2_ref[:].astype(jnp.bfloat16), w_ref[:],
        dimension_numbers=(((1,), (0,)), ((), ())),
        preferred_element_type=jnp.float32,
    )
    out_ref[:] = jnp.maximum(a + b_ref[0], 0.0).reshape(1, _NBB, _O)


def _tc_matmul(e2_h, W, b2d, half, prev=None):
    kwargs = {}
    operands = [e2_h, W, b2d]
    in_specs = [
        pl.BlockSpec((_NBB, _O), lambda f, i: (f * _NBGRID + i, 0)),
        pl.BlockSpec((_O, _O), lambda f, i: (0, 0)),
        pl.BlockSpec((1, _O), lambda f, i: (0, 0)),
    ]
    if prev is not None:
        operands.append(prev)
        kwargs["input_output_aliases"] = {3: 0}
        in_specs.append(pl.BlockSpec((1, 8, _O), lambda f, i: (0, 0, 0)))

    def body(*refs):
        _mm_body(refs[0], refs[1], refs[2], refs[-1])

    return pl.pallas_call(
        body,
        grid=(_FPH, _NBGRID),
        in_specs=in_specs,
        out_specs=pl.BlockSpec(
            (1, _NBB, _O), lambda f, i, h=half: (f + h * _FPH, i, 0)
        ),
        out_shape=jax.ShapeDtypeStruct((_F, _B, _O), jnp.float32),
        **kwargs,
    )(*operands)


def kernel(x, emb_table, W, b):
    idx = x.astype(jnp.int32).T.reshape(_BF)
    tab128 = jnp.pad(emb_table, ((0, 0), (0, _O - _E)))
    wbf = jnp.concatenate([W.T, jnp.zeros((_O - _E, _O), W.dtype)], axis=0)
    wbf = wbf.astype(jnp.bfloat16)
    b2d = b.reshape(1, _O)
    e2 = [_sc_gather_halves[h](tab128, idx) for h in range(_NHALF)]
    out = _tc_matmul(e2[0], wbf, b2d, 0)
    for h in range(1, _NHALF):
        out = _tc_matmul(e2[h], wbf, b2d, h, prev=out)
    return jnp.transpose(out, (1, 0, 2))


# speedup vs baseline: 1.1809x; 1.0278x over previous
import functools

import jax
import jax.numpy as jnp
from jax import lax
from jax.experimental import pallas as pl
from jax.experimental.pallas import tpu as pltpu
from jax.experimental.pallas import tpu_sc as plsc

_V = 100000
_E = 64
_O = 128
_B = 16384
_F = 26
_BF = _B * _F

_NHALF = 2
_FH = _BF // _NHALF

_NC = 2
_NS = 16
_NW = _NC * _NS
_FPW = _FH // _NW
_CH = 128
_NCHUNK = _FPW // _CH
_NBUF = 4

_NBB = 8192
_FPH = _F // _NHALF
_NBGRID = _B // _NBB

_sc_mesh = plsc.VectorSubcoreMesh(core_axis_name="c", subcore_axis_name="s")


def _make_sc_gather(half):
    @functools.partial(
        pl.kernel,
        mesh=_sc_mesh,
        out_type=jax.ShapeDtypeStruct((_FH, _O), jnp.float32),
        scratch_types=[
            pltpu.VMEM((_FPW,), jnp.int32),
            pltpu.VMEM((_NBUF, _CH, _O), jnp.float32),
            pltpu.SemaphoreType.DMA,
            pltpu.SemaphoreType.DMA,
        ],
        name=f"sc_gather_h{half}",
    )
    def _sc_gather(tab_hbm, idx_hbm, e2_hbm, idx_v, rows_v, gsem, ssem):
        wid = lax.axis_index("s") * _NC + lax.axis_index("c")
        fbase = wid * _FPW
        pltpu.sync_copy(idx_hbm.at[pl.ds(half * _FH + fbase, _FPW)], idx_v)

        def outer(jo, carry):
            j0 = jo * _NBUF
            gathers = []
            for bi in range(_NBUF):
                ids = idx_v.at[pl.ds((j0 + bi) * _CH, _CH)]
                gathers.append(pltpu.async_copy(tab_hbm.at[ids], rows_v.at[bi], gsem))
            scatters = []
            for bi in range(_NBUF):
                gathers[bi].wait()
                dst = e2_hbm.at[pl.ds(fbase + (j0 + bi) * _CH, _CH)]
                scatters.append(pltpu.async_copy(rows_v.at[bi], dst, ssem))
            for s in scatters:
                s.wait()
            return carry

        lax.fori_loop(0, _NCHUNK // _NBUF, outer, 0)

    return _sc_gather


_sc_gather_halves = [_make_sc_gather(h) for h in range(_NHALF)]


def _mm_body(e2_ref, w_ref, b_ref, out_ref):
    a = lax.dot_general(
        e2_ref[:].astype(jnp.bfloat16), w_ref[:],
        dimension_numbers=(((1,), (0,)), ((), ())),
        preferred_element_type=jnp.float32,
    )
    out_ref[:] = jnp.maximum(a + b_ref[0], 0.0).reshape(1, _NBB, _O)


def _tc_matmul(e2_h, W, b2d, half, prev=None):
    kwargs = {}
    operands = [e2_h, W, b2d]
    in_specs = [
        pl.BlockSpec((_NBB, _O), lambda f, i: (f * _NBGRID + i, 0)),
        pl.BlockSpec((_O, _O), lambda f, i: (0, 0)),
        pl.BlockSpec((1, _O), lambda f, i: (0, 0)),
    ]
    if prev is not None:
        operands.append(prev)
        kwargs["input_output_aliases"] = {3: 0}
        in_specs.append(pl.BlockSpec((1, 8, _O), lambda f, i: (0, 0, 0)))

    def body(*refs):
        _mm_body(refs[0], refs[1], refs[2], refs[-1])

    return pl.pallas_call(
        body,
        grid=(_FPH, _NBGRID),
        in_specs=in_specs,
        out_specs=pl.BlockSpec(
            (1, _NBB, _O), lambda f, i, h=half: (f + h * _FPH, i, 0)
        ),
        out_shape=jax.ShapeDtypeStruct((_F, _B, _O), jnp.float32),
        **kwargs,
    )(*operands)


def kernel(x, emb_table, W, b):
    idx = x.astype(jnp.int32).T.reshape(_BF)
    tab128 = jnp.pad(emb_table, ((0, 0), (0, _O - _E)))
    wbf = jnp.concatenate([W.T, jnp.zeros((_O - _E, _O), W.dtype)], axis=0)
    wbf = wbf.astype(jnp.bfloat16)
    b2d = b.reshape(1, _O)
    e2 = [_sc_gather_halves[h](tab128, idx) for h in range(_NHALF)]
    out = _tc_matmul(e2[0], wbf, b2d, 0)
    for h in range(1, _NHALF):
        out = _tc_matmul(e2[h], wbf, b2d, h, prev=out)
    return jnp.transpose(out, (1, 0, 2))


# speedup vs baseline: 1.1834x; 1.0021x over previous
import functools

import jax
import jax.numpy as jnp
from jax import lax
from jax.experimental import pallas as pl
from jax.experimental.pallas import tpu as pltpu
from jax.experimental.pallas import tpu_sc as plsc

_V = 100000
_E = 64
_O = 128
_B = 16384
_F = 26
_BF = _B * _F

_SPLITS = [(0, 14), (14, 12)]

_NC = 2
_NS = 16
_NW = _NC * _NS
_CH = 128
_NBUF = 4

_NBB = 8192
_NBGRID = _B // _NBB

_sc_mesh = plsc.VectorSubcoreMesh(core_axis_name="c", subcore_axis_name="s")


def _make_sc_gather(f_lo, nf):
    fh = nf * _B
    fpw = fh // _NW
    nchunk = fpw // _CH

    @functools.partial(
        pl.kernel,
        mesh=_sc_mesh,
        out_type=jax.ShapeDtypeStruct((fh, _O), jnp.float32),
        scratch_types=[
            pltpu.VMEM((fpw,), jnp.int32),
            pltpu.VMEM((_NBUF, _CH, _O), jnp.float32),
            pltpu.SemaphoreType.DMA,
            pltpu.SemaphoreType.DMA,
        ],
        name=f"sc_gather_f{f_lo}",
    )
    def _sc_gather(tab_hbm, idx_hbm, e2_hbm, idx_v, rows_v, gsem, ssem):
        wid = lax.axis_index("s") * _NC + lax.axis_index("c")
        fbase = wid * fpw
        pltpu.sync_copy(idx_hbm.at[pl.ds(f_lo * _B + fbase, fpw)], idx_v)

        def outer(jo, carry):
            j0 = jo * _NBUF
            gathers = []
            for bi in range(_NBUF):
                ids = idx_v.at[pl.ds((j0 + bi) * _CH, _CH)]
                gathers.append(pltpu.async_copy(tab_hbm.at[ids], rows_v.at[bi], gsem))
            scatters = []
            for bi in range(_NBUF):
                gathers[bi].wait()
                dst = e2_hbm.at[pl.ds(fbase + (j0 + bi) * _CH, _CH)]
                scatters.append(pltpu.async_copy(rows_v.at[bi], dst, ssem))
            for s in scatters:
                s.wait()
            return carry

        lax.fori_loop(0, nchunk // _NBUF, outer, 0)

    return _sc_gather


_sc_gather_parts = [_make_sc_gather(f_lo, nf) for f_lo, nf in _SPLITS]


def _mm_body(e2_ref, w_ref, b_ref, out_ref):
    a = lax.dot_general(
        e2_ref[:].astype(jnp.bfloat16), w_ref[:],
        dimension_numbers=(((1,), (0,)), ((), ())),
        preferred_element_type=jnp.float32,
    )
    out_ref[:] = jnp.maximum(a + b_ref[0], 0.0).reshape(1, _NBB, _O)


def _tc_matmul(e2_h, W, b2d, f_lo, nf, prev=None):
    kwargs = {}
    operands = [e2_h, W, b2d]
    in_specs = [
        pl.BlockSpec((_NBB, _O), lambda f, i: (f * _NBGRID + i, 0)),
        pl.BlockSpec((_O, _O), lambda f, i: (0, 0)),
        pl.BlockSpec((1, _O), lambda f, i: (0, 0)),
    ]
    if prev is not None:
        operands.append(prev)
        kwargs["input_output_aliases"] = {3: 0}
        in_specs.append(pl.BlockSpec((1, 8, _O), lambda f, i: (0, 0, 0)))

    def body(*refs):
        _mm_body(refs[0], refs[1], refs[2], refs[-1])

    return pl.pallas_call(
        body,
        grid=(nf, _NBGRID),
        in_specs=in_specs,
        out_specs=pl.BlockSpec(
            (1, _NBB, _O), lambda f, i: (f + f_lo, i, 0)
        ),
        out_shape=jax.ShapeDtypeStruct((_F, _B, _O), jnp.float32),
        **kwargs,
    )(*operands)


def kernel(x, emb_table, W, b):
    idx = x.astype(jnp.int32).T.reshape(_BF)
    tab128 = jnp.pad(emb_table, ((0, 0), (0, _O - _E)))
    wbf = jnp.concatenate([W.T, jnp.zeros((_O - _E, _O), W.dtype)], axis=0)
    wbf = wbf.astype(jnp.bfloat16)
    b2d = b.reshape(1, _O)
    e2 = [g(tab128, idx) for g in _sc_gather_parts]
    out = _tc_matmul(e2[0], wbf, b2d, *_SPLITS[0])
    for h in range(1, len(_SPLITS)):
        out = _tc_matmul(e2[h], wbf, b2d, *_SPLITS[h], prev=out)
    return jnp.transpose(out, (1, 0, 2))
